# Initial kernel scaffold; baseline (speedup 1.0000x reference)
import jax
import jax.numpy as jnp
from jax.experimental import pallas as pl


def kernel(x, edge_index, edge_attr, batch, Wl1, Wr1, We1, att1, b1, Wl2, Wr2, We2, att2, b2, Wih, Whh, bih, bhh, Wc1, bc1, Wc2, bc2):
    raise NotImplementedError("write your pallas kernel here")


# speedup vs baseline: 8.4545x; 8.4545x over previous
import functools

import jax
import jax.numpy as jnp
import numpy as np
from jax import lax
from jax.experimental import pallas as pl
from jax.experimental.pallas import tpu as pltpu
from jax.experimental.pallas import tpu_sc as plsc

_N = 10000
_E = 160000
_G = 64
_NC = 2
_NS = 16
_NW = _NC * _NS
_EPW = _E // _NW
_C = 40
_NPAD = 10240
_RPT = _NPAD // _NS

_call = pl.pallas_call


def _sc_mesh():
    return plsc.VectorSubcoreMesh(core_axis_name="c", subcore_axis_name="s")


def _gather_pair_sc(ta, tb, src, dst, F):
    def body(ta_h, tb_h, src_h, dst_h, oa_h, ob_h, idx_v, rows_v, sem):
        c = lax.axis_index("c")
        s = lax.axis_index("s")
        base = (s * _NC + c) * _EPW

        def it(i, carry):
            off = base + i * _C
            pltpu.sync_copy(src_h.at[pl.ds(off, _C)], idx_v)
            pltpu.async_copy(ta_h.at[idx_v], rows_v, sem).wait()
            pltpu.sync_copy(rows_v, oa_h.at[pl.ds(off, _C)])
            pltpu.sync_copy(dst_h.at[pl.ds(off, _C)], idx_v)
            pltpu.async_copy(tb_h.at[idx_v], rows_v, sem).wait()
            pltpu.sync_copy(rows_v, ob_h.at[pl.ds(off, _C)])
            return carry

        lax.fori_loop(0, _EPW // _C, it, 0)

    k = pl.kernel(
        body,
        out_type=[jax.ShapeDtypeStruct((_E, F), jnp.float32)] * 2,
        mesh=_sc_mesh(),
        scratch_types=[
            pltpu.VMEM((_C,), jnp.int32),
            pltpu.VMEM((_C, F), jnp.float32),
            pltpu.SemaphoreType.DMA,
        ],
    )
    return k(ta, tb, src, dst)


def _scatter_add_sc(rows, dst, zeros, D):
    def body(rows_h, dst_h, z_h, out_h, idx_v, rows_v, acc_sh):
        c = lax.axis_index("c")
        s = lax.axis_index("s")
        base = (s * _NC + c) * _EPW
        pltpu.sync_copy(z_h.at[pl.ds(s * _RPT, _RPT)],
                        acc_sh.at[pl.ds(s * _RPT, _RPT)])
        plsc.subcore_barrier()

        def it(i, carry):
            off = base + i * _C
            pltpu.sync_copy(dst_h.at[pl.ds(off, _C)], idx_v)
            pltpu.sync_copy(rows_h.at[pl.ds(off, _C)], rows_v)
            pltpu.sync_copy(rows_v, acc_sh.at[idx_v], add=True)
            return carry

        lax.fori_loop(0, _EPW // _C, it, 0)
        plsc.subcore_barrier()
        pltpu.sync_copy(acc_sh.at[pl.ds(s * _RPT, _RPT)],
                        out_h.at[c, pl.ds(s * _RPT, _RPT)])

    k = pl.kernel(
        body,
        out_type=jax.ShapeDtypeStruct((2, _NPAD, D), jnp.float32),
        mesh=_sc_mesh(),
        scratch_types=[
            pltpu.VMEM((_C,), jnp.int32),
            pltpu.VMEM((_C, D), jnp.float32),
            pltpu.VMEM_SHARED((_NPAD, D), jnp.float32),
        ],
    )
    out = k(rows, dst, zeros)
    return out[:, :_N]


def _proj_pair(xp, Wl, Wr, K, F, blk=1000):
    def body(x_ref, wl_ref, wr_ref, ol_ref, or_ref):
        xv = x_ref[...]
        ol_ref[...] = jnp.dot(xv, wl_ref[...], preferred_element_type=jnp.float32, precision=lax.Precision.HIGHEST)
        or_ref[...] = jnp.dot(xv, wr_ref[...], preferred_element_type=jnp.float32, precision=lax.Precision.HIGHEST)

    return _call(
        body,
        grid=(_N // blk,),
        in_specs=[
            pl.BlockSpec((blk, K), lambda i: (i, 0)),
            pl.BlockSpec((K, F), lambda i: (0, 0)),
            pl.BlockSpec((K, F), lambda i: (0, 0)),
        ],
        out_specs=[pl.BlockSpec((blk, F), lambda i: (i, 0))] * 2,
        out_shape=[jax.ShapeDtypeStruct((_N, F), jnp.float32)] * 2,
    )(xp, Wl, Wr)


def _edge1(xls, xrd, ea, We, attf, S8, T8, P128, blk=1000):
    def body(xls_ref, xrd_ref, ea_ref, we_ref, att_ref, s8_ref, t8_ref,
             p16_ref, ca_ref, cb_ref, dn_ref):
        ea_v = ea_ref[...]
        ef = ea_v[:, 0:1] * we_ref[0:1, :] + ea_v[:, 1:2] * we_ref[1:2, :]
        xlv = xls_ref[...]
        m = xlv + xrd_ref[...] + ef
        m = jnp.where(m > 0, m, 0.2 * m)
        lg = jnp.dot(m * att_ref[...], s8_ref[...],
                     preferred_element_type=jnp.float32, precision=lax.Precision.HIGHEST)
        ex8 = jnp.exp(lg)
        exb = jnp.dot(ex8, t8_ref[...], preferred_element_type=jnp.float32, precision=lax.Precision.HIGHEST)
        contrib = exb * xlv
        ca_ref[...] = contrib[:, :128]
        cb_ref[...] = contrib[:, 128:]
        dn_ref[...] = jnp.dot(ex8, p16_ref[...],
                              preferred_element_type=jnp.float32, precision=lax.Precision.HIGHEST)

    return _call(
        body,
        grid=(_E // blk,),
        in_specs=[
            pl.BlockSpec((blk, 256), lambda i: (i, 0)),
            pl.BlockSpec((blk, 256), lambda i: (i, 0)),
            pl.BlockSpec((blk, 2), lambda i: (i, 0)),
            pl.BlockSpec((2, 256), lambda i: (0, 0)),
            pl.BlockSpec((1, 256), lambda i: (0, 0)),
            pl.BlockSpec((256, 8), lambda i: (0, 0)),
            pl.BlockSpec((8, 256), lambda i: (0, 0)),
            pl.BlockSpec((8, 128), lambda i: (0, 0)),
        ],
        out_specs=[
            pl.BlockSpec((blk, 128), lambda i: (i, 0)),
            pl.BlockSpec((blk, 128), lambda i: (i, 0)),
            pl.BlockSpec((blk, 128), lambda i: (i, 0)),
        ],
        out_shape=[
            jax.ShapeDtypeStruct((_E, 128), jnp.float32),
            jax.ShapeDtypeStruct((_E, 128), jnp.float32),
            jax.ShapeDtypeStruct((_E, 128), jnp.float32),
        ],
    )(xls, xrd, ea, We, attf, S8, T8, P128)


def _node1(xl1, xr1, a0, a1, b0, b1, d0, d1, e0, e1, We, attf, S8, T8, P128,
           T128, b1f, Wl2, Wr2, blk=1000):
    def body(xl_ref, xr_ref, a0_ref, a1_ref, b0_ref, b1_ref, d0_ref, d1_ref,
             e0_ref, e1_ref, we_ref, att_ref, s8_ref, t8_ref, p16_ref,
             t16_ref, bias_ref, wl2_ref, wr2_ref, ot_ref):
        xlv = xl_ref[...]
        eacc = e0_ref[...] + e1_ref[...]
        cnt = jnp.maximum(eacc[:, 2:3], 1.0)
        la0 = eacc[:, 0:1] / cnt
        la1 = eacc[:, 1:2] / cnt
        efl = la0 * we_ref[0:1, :] + la1 * we_ref[1:2, :]
        ml = xlv + xr_ref[...] + efl
        ml = jnp.where(ml > 0, ml, 0.2 * ml)
        exl8 = jnp.exp(jnp.dot(ml * att_ref[...], s8_ref[...],
                               preferred_element_type=jnp.float32, precision=lax.Precision.HIGHEST))
        exlb = jnp.dot(exl8, t8_ref[...], preferred_element_type=jnp.float32, precision=lax.Precision.HIGHEST)
        den16 = d0_ref[...] + d1_ref[...] + jnp.dot(
            exl8, p16_ref[...], preferred_element_type=jnp.float32, precision=lax.Precision.HIGHEST)
        denb = jnp.dot(den16, t16_ref[...], preferred_element_type=jnp.float32, precision=lax.Precision.HIGHEST)
        num = jnp.concatenate(
            [a0_ref[...] + a1_ref[...], b0_ref[...] + b1_ref[...]], axis=1)
        num = num + exlb * xlv
        out = num / (denb + 1e-16) + bias_ref[...]
        h1 = jnp.where(out > 0, out, jnp.exp(out) - 1.0)
        ot_ref[...] = jnp.concatenate(
            [jnp.dot(h1, wl2_ref[...], preferred_element_type=jnp.float32, precision=lax.Precision.HIGHEST),
             jnp.dot(h1, wr2_ref[...], preferred_element_type=jnp.float32, precision=lax.Precision.HIGHEST)],
            axis=1)

    nb = lambda i: (i, 0)
    z = lambda i: (0, 0)
    return _call(
        body,
        grid=(_N // blk,),
        in_specs=[
            pl.BlockSpec((blk, 256), nb), pl.BlockSpec((blk, 256), nb),
            pl.BlockSpec((blk, 128), nb), pl.BlockSpec((blk, 128), nb),
            pl.BlockSpec((blk, 128), nb), pl.BlockSpec((blk, 128), nb),
            pl.BlockSpec((blk, 128), nb), pl.BlockSpec((blk, 128), nb),
            pl.BlockSpec((blk, 128), nb), pl.BlockSpec((blk, 128), nb),
            pl.BlockSpec((2, 256), z), pl.BlockSpec((1, 256), z),
            pl.BlockSpec((256, 8), z), pl.BlockSpec((8, 256), z),
            pl.BlockSpec((8, 128), z), pl.BlockSpec((128, 256), z),
            pl.BlockSpec((1, 256), z),
            pl.BlockSpec((256, 64), z), pl.BlockSpec((256, 64), z),
        ],
        out_specs=[pl.BlockSpec((blk, 128), nb)],
        out_shape=[jax.ShapeDtypeStruct((_N, 128), jnp.float32)],
    )(xl1, xr1, a0, a1, b0, b1, d0, d1, e0, e1, We, attf, S8, T8, P128, T128,
      b1f, Wl2, Wr2)[0]


def _edge2(xls, xrd, ea, We, attf, u64, blk=1000):
    def body(xls_ref, xrd_ref, ea_ref, we_ref, att_ref, u64_ref, c_ref):
        ea_v = ea_ref[...]
        ef = ea_v[:, 0:1] * we_ref[0:1, :] + ea_v[:, 1:2] * we_ref[1:2, :]
        xlv = xls_ref[:, 0:64]
        m = xlv + xrd_ref[:, 64:128] + ef
        m = jnp.where(m > 0, m, 0.2 * m)
        ex = jnp.exp(jnp.sum(m * att_ref[...], axis=1, keepdims=True))
        c_ref[...] = jnp.concatenate([ex * xlv, ex * u64_ref[...]], axis=1)

    nb = lambda i: (i, 0)
    z = lambda i: (0, 0)
    return _call(
        body,
        grid=(_E // blk,),
        in_specs=[
            pl.BlockSpec((blk, 128), nb), pl.BlockSpec((blk, 128), nb),
            pl.BlockSpec((blk, 2), nb),
            pl.BlockSpec((2, 64), z), pl.BlockSpec((1, 64), z),
            pl.BlockSpec((1, 64), z),
        ],
        out_specs=[pl.BlockSpec((blk, 128), nb)],
        out_shape=[jax.ShapeDtypeStruct((_E, 128), jnp.float32)],
    )(xls, xrd, ea, We, attf, u64)[0]


def _node2(t2, c0, c1, e0, e1, We, attf, b2f, blk=1000):
    def body(t_ref, c0_ref, c1_ref, e0_ref, e1_ref,
             we_ref, att_ref, bias_ref, oh_ref):
        xlv = t_ref[:, 0:64]
        eacc = e0_ref[...] + e1_ref[...]
        cnt = jnp.maximum(eacc[:, 2:3], 1.0)
        la0 = eacc[:, 0:1] / cnt
        la1 = eacc[:, 1:2] / cnt
        efl = la0 * we_ref[0:1, :] + la1 * we_ref[1:2, :]
        ml = xlv + t_ref[:, 64:128] + efl
        ml = jnp.where(ml > 0, ml, 0.2 * ml)
        exl = jnp.exp(jnp.sum(ml * att_ref[...], axis=1, keepdims=True))
        cacc = c0_ref[...] + c1_ref[...]
        dent = cacc[:, 64:65] + exl
        num = cacc[:, 0:64] + exl * xlv
        out = num / (dent + 1e-16) + bias_ref[...]
        oh_ref[...] = jnp.where(out > 0, out, jnp.exp(out) - 1.0)

    nb = lambda i: (i, 0)
    z = lambda i: (0, 0)
    return _call(
        body,
        grid=(_N // blk,),
        in_specs=[
            pl.BlockSpec((blk, 128), nb),
            pl.BlockSpec((blk, 128), nb), pl.BlockSpec((blk, 128), nb),
            pl.BlockSpec((blk, 128), nb), pl.BlockSpec((blk, 128), nb),
            pl.BlockSpec((2, 64), z), pl.BlockSpec((1, 64), z),
            pl.BlockSpec((1, 64), z),
        ],
        out_specs=[pl.BlockSpec((blk, 64), nb)],
        out_shape=[jax.ShapeDtypeStruct((_N, 64), jnp.float32)],
    )(t2, c0, c1, e0, e1, We, attf, b2f)[0]


def _sig(x):
    return 1.0 / (1.0 + jnp.exp(-x))


def _head(h2, bt, WihT, WhhT, bih, bhh, Wc1, bc1, Wc2p, bc2p):
    nchunk = _N // 1000

    def body(h2_ref, bt_ref, wih_ref, whh_ref, bih_ref, bhh_ref, wc1_ref,
             bc1_ref, wc2_ref, bc2_ref, o_ref, gs_ref, ct_ref, gi_ref):
        t = pl.program_id(0)

        @pl.when(t == 0)
        def _init():
            gs_ref[...] = jnp.zeros((_G, 64), jnp.float32)
            ct_ref[...] = jnp.zeros((_G, 1), jnp.float32)

        bb = bt_ref[0]
        gid = lax.broadcasted_iota(jnp.int32, (_G, 1000), 0)
        oh = (gid == bb).astype(jnp.float32)
        gs_ref[...] += jnp.dot(oh, h2_ref[...],
                               preferred_element_type=jnp.float32, precision=lax.Precision.HIGHEST)
        ct_ref[...] += jnp.sum(oh, axis=1, keepdims=True)

        @pl.when(t == nchunk - 1)
        def _finish():
            g = gs_ref[...] / jnp.maximum(ct_ref[...], 1.0)
            gi_ref[...] = jnp.dot(
                g, wih_ref[...], preferred_element_type=jnp.float32, precision=lax.Precision.HIGHEST
            ) + bih_ref[...]

            def gru(i, h):
                gi = gi_ref[pl.ds(i, 1), :]
                gh = jnp.dot(h, whh_ref[...],
                             preferred_element_type=jnp.float32, precision=lax.Precision.HIGHEST) + bhh_ref[...]
                r = _sig(gi[:, 0:64] + gh[:, 0:64])
                zz = _sig(gi[:, 64:128] + gh[:, 64:128])
                nt = jnp.tanh(gi[:, 128:192] + r * gh[:, 128:192])
                return (1.0 - zz) * nt + zz * h

            h = lax.fori_loop(0, _G, gru, jnp.zeros((1, 64), jnp.float32))
            z1 = jnp.maximum(
                jnp.dot(h, wc1_ref[...], preferred_element_type=jnp.float32, precision=lax.Precision.HIGHEST)
                + bc1_ref[...], 0.0)
            o_ref[...] = _sig(
                jnp.dot(z1, wc2_ref[...], preferred_element_type=jnp.float32, precision=lax.Precision.HIGHEST)
                + bc2_ref[...])

    nb = lambda i: (i, 0)
    z = lambda i: (0, 0)
    return _call(
        body,
        grid=(nchunk,),
        in_specs=[
            pl.BlockSpec((1000, 64), nb),
            pl.BlockSpec((1, 1, 1000), lambda i: (i, 0, 0)),
            pl.BlockSpec((64, 192), z), pl.BlockSpec((64, 192), z),
            pl.BlockSpec((1, 192), z), pl.BlockSpec((1, 192), z),
            pl.BlockSpec((64, 32), z), pl.BlockSpec((1, 32), z),
            pl.BlockSpec((32, 8), z), pl.BlockSpec((1, 8), z),
        ],
        out_specs=[pl.BlockSpec((1, 8), z)],
        out_shape=[jax.ShapeDtypeStruct((1, 8), jnp.float32)],
        scratch_shapes=[
            pltpu.VMEM((_G, 64), jnp.float32),
            pltpu.VMEM((_G, 1), jnp.float32),
            pltpu.VMEM((_G, 192), jnp.float32),
        ],
    )(h2, bt, WihT, WhhT, bih, bhh, Wc1, bc1, Wc2p, bc2p)[0]


_HSEL = np.arange(256) // 64
_S8 = np.zeros((256, 8), np.float32)
_S8[np.arange(256), _HSEL] = 1.0
_T8 = np.ascontiguousarray(_S8.T)
_P128 = np.zeros((8, 128), np.float32)
for _i in range(4):
    _P128[_i, _i] = 1.0
_T128 = np.zeros((128, 256), np.float32)
for _h in range(4):
    _T128[_h, _h * 64:(_h + 1) * 64] = 1.0
_U64 = np.zeros((1, 64), np.float32)
_U64[0, 0] = 1.0


def kernel(x, edge_index, edge_attr, batch, Wl1, Wr1, We1, att1, b1, Wl2,
           Wr2, We2, att2, b2, Wih, Whh, bih, bhh, Wc1, bc1, Wc2, bc2):
    f32 = jnp.float32
    src = edge_index[0]
    dst = edge_index[1]

    xp = jnp.pad(x, ((0, 0), (0, 3)))
    Wl1p = jnp.pad(Wl1, ((0, 3), (0, 0)))
    Wr1p = jnp.pad(Wr1, ((0, 3), (0, 0)))
    attf1 = att1.reshape(1, 256)
    attf2 = att2.reshape(1, 64)
    b1f = b1.reshape(1, 256)
    b2f = b2.reshape(1, 64)
    S8 = jnp.asarray(_S8)
    T8 = jnp.asarray(_T8)
    P128 = jnp.asarray(_P128)
    T128 = jnp.asarray(_T128)
    u64 = jnp.asarray(_U64)
    z128 = jnp.zeros((_NPAD, 128), f32)

    rows_ea = jnp.concatenate(
        [edge_attr, jnp.ones((_E, 1), f32), jnp.zeros((_E, 125), f32)], axis=1)
    eacc = _scatter_add_sc(rows_ea, dst, z128, 128)

    xl1, xr1 = _proj_pair(xp, Wl1p, Wr1p, 8, 256)
    xls1, xrd1 = _gather_pair_sc(xl1, xr1, src, dst, 256)
    cA, cB, dn1 = _edge1(xls1, xrd1, edge_attr, We1, attf1, S8, T8, P128)
    numA = _scatter_add_sc(cA, dst, z128, 128)
    numB = _scatter_add_sc(cB, dst, z128, 128)
    den1 = _scatter_add_sc(dn1, dst, z128, 128)
    t2 = _node1(xl1, xr1, numA[0], numA[1], numB[0], numB[1], den1[0],
                den1[1], eacc[0], eacc[1], We1, attf1, S8, T8, P128, T128,
                b1f, Wl2, Wr2)

    xls2, xrd2 = _gather_pair_sc(t2, t2, src, dst, 128)
    c2 = _edge2(xls2, xrd2, edge_attr, We2, attf2, u64)
    acc2 = _scatter_add_sc(c2, dst, z128, 128)
    h2 = _node2(t2, acc2[0], acc2[1], eacc[0], eacc[1], We2, attf2, b2f)

    WihT = Wih.T
    WhhT = Whh.T
    Wc2p = jnp.pad(Wc2, ((0, 0), (0, 7)))
    bc2p = jnp.pad(bc2.reshape(1, 1), ((0, 0), (0, 7)))
    out8 = _head(h2, batch.reshape(10, 1, 1000), WihT, WhhT, bih.reshape(1, 192),
                 bhh.reshape(1, 192), Wc1, bc1.reshape(1, 32), Wc2p, bc2p)
    return out8[0:1, 0:1]


# speedup vs baseline: 14.0742x; 1.6647x over previous
import functools

import jax
import jax.numpy as jnp
import numpy as np
from jax import lax
from jax.experimental import pallas as pl
from jax.experimental.pallas import tpu as pltpu
from jax.experimental.pallas import tpu_sc as plsc

_N = 10000
_E = 160000
_G = 64
_NC = 2
_NS = 16
_NW = _NC * _NS
_EPW = _E // _NW
_C = 40
_NPAD = 10240
_RPT = _NPAD // _NS

_call = pl.pallas_call


def _sc_mesh():
    return plsc.VectorSubcoreMesh(core_axis_name="c", subcore_axis_name="s")


_CHUNKS = [(i * 120, 120) for i in range(41)] + [(4920, 80)]
_NBUF = 3


def _gather_pair_sc(ta, tb, src, dst, F):
    def body(ta_h, tb_h, src_h, dst_h, oa_h, ob_h, idxs_v, idxd_v,
             b0, b1, b2, gs0, gs1, gs2, ws0, ws1, ws2):
        c = lax.axis_index("c")
        s = lax.axis_index("s")
        base = (s * _NC + c) * _EPW
        pltpu.sync_copy(src_h.at[pl.ds(base, _EPW)], idxs_v)
        pltpu.sync_copy(dst_h.at[pl.ds(base, _EPW)], idxd_v)
        bufs = (b0, b1, b2)
        gsems = (gs0, gs1, gs2)
        wsems = (ws0, ws1, ws2)

        steps = []
        for tab, out, idxv in ((ta_h, oa_h, idxs_v), (tb_h, ob_h, idxd_v)):
            for off, sz in _CHUNKS:
                steps.append((tab, out, idxv, off, sz))

        pend_g = [None] * _NBUF
        pend_w = [None] * _NBUF
        meta = [None] * _NBUF
        for j, (tab, out, idxv, off, sz) in enumerate(steps):
            b = j % _NBUF
            if pend_w[b] is not None:
                pend_w[b].wait()
                pend_w[b] = None
            pend_g[b] = pltpu.async_copy(
                tab.at[idxv.at[pl.ds(off, sz)]],
                bufs[b].at[pl.ds(0, sz)], gsems[b])
            meta[b] = (out, off, sz)
            bp = (j + 1) % _NBUF
            if j >= _NBUF - 1 and pend_g[bp] is not None:
                pend_g[bp].wait()
                pend_g[bp] = None
                outp, offp, szp = meta[bp]
                pend_w[bp] = pltpu.async_copy(
                    bufs[bp].at[pl.ds(0, szp)],
                    outp.at[pl.ds(base + offp, szp)], wsems[bp])
        for b in range(_NBUF):
            if pend_g[b] is not None:
                pend_g[b].wait()
                outp, offp, szp = meta[b]
                pend_w[b] = pltpu.async_copy(
                    bufs[b].at[pl.ds(0, szp)],
                    outp.at[pl.ds(base + offp, szp)], wsems[b])
            if pend_w[b] is not None:
                pend_w[b].wait()

    k = pl.kernel(
        body,
        out_type=[jax.ShapeDtypeStruct((_E, F), jnp.float32)] * 2,
        mesh=_sc_mesh(),
        scratch_types=[
            pltpu.VMEM((_EPW,), jnp.int32),
            pltpu.VMEM((_EPW,), jnp.int32),
            pltpu.VMEM((120, F), jnp.float32),
            pltpu.VMEM((120, F), jnp.float32),
            pltpu.VMEM((120, F), jnp.float32),
            pltpu.SemaphoreType.DMA, pltpu.SemaphoreType.DMA,
            pltpu.SemaphoreType.DMA, pltpu.SemaphoreType.DMA,
            pltpu.SemaphoreType.DMA, pltpu.SemaphoreType.DMA,
        ],
    )
    return k(ta, tb, src, dst)


def _scatter_add_sc(rows, dst, zeros, D):
    def body(rows_h, dst_h, z_h, out_h, i0, i1, it_, r0, r1, s0, s1, acc_sh):
        c = lax.axis_index("c")
        s = lax.axis_index("s")
        base = (s * _NC + c) * _EPW
        pltpu.sync_copy(z_h.at[pl.ds(s * _RPT, _RPT)],
                        acc_sh.at[pl.ds(s * _RPT, _RPT)])
        plsc.subcore_barrier()

        ibufs = (i0, i1)
        rbufs = (r0, r1)
        sems = (s0, s1)
        nch = len(_CHUNKS)

        def issue(k):
            off, sz = _CHUNKS[k]
            b = k % 2
            ib = it_ if sz != 120 else ibufs[b]
            icp = pltpu.async_copy(dst_h.at[pl.ds(base + off, sz)], ib,
                                   sems[b])
            rcp = pltpu.async_copy(rows_h.at[pl.ds(base + off, sz)],
                                   rbufs[b].at[pl.ds(0, sz)], sems[b])
            return icp, rcp, ib, sz

        pend = [None, None]
        pend[0] = issue(0)
        for k in range(nch):
            b = k % 2
            if k + 1 < nch:
                pend[(k + 1) % 2] = issue(k + 1)
            icp, rcp, ib, sz = pend[b]
            icp.wait()
            rcp.wait()
            pltpu.sync_copy(rbufs[b].at[pl.ds(0, sz)], acc_sh.at[ib],
                            add=True)

        plsc.subcore_barrier()
        pltpu.sync_copy(acc_sh.at[pl.ds(s * _RPT, _RPT)],
                        out_h.at[c, pl.ds(s * _RPT, _RPT)])

    k = pl.kernel(
        body,
        out_type=jax.ShapeDtypeStruct((2, _NPAD, D), jnp.float32),
        mesh=_sc_mesh(),
        scratch_types=[
            pltpu.VMEM((120,), jnp.int32),
            pltpu.VMEM((120,), jnp.int32),
            pltpu.VMEM((80,), jnp.int32),
            pltpu.VMEM((120, D), jnp.float32),
            pltpu.VMEM((120, D), jnp.float32),
            pltpu.SemaphoreType.DMA, pltpu.SemaphoreType.DMA,
            pltpu.VMEM_SHARED((_NPAD, D), jnp.float32),
        ],
    )
    out = k(rows, dst, zeros)
    return out[:, :_N]


def _proj_pair(xp, Wl, Wr, K, F, blk=1000):
    def body(x_ref, wl_ref, wr_ref, ol_ref, or_ref):
        xv = x_ref[...]
        ol_ref[...] = jnp.dot(xv, wl_ref[...], preferred_element_type=jnp.float32, precision=lax.Precision.HIGHEST)
        or_ref[...] = jnp.dot(xv, wr_ref[...], preferred_element_type=jnp.float32, precision=lax.Precision.HIGHEST)

    return _call(
        body,
        grid=(_N // blk,),
        in_specs=[
            pl.BlockSpec((blk, K), lambda i: (i, 0)),
            pl.BlockSpec((K, F), lambda i: (0, 0)),
            pl.BlockSpec((K, F), lambda i: (0, 0)),
        ],
        out_specs=[pl.BlockSpec((blk, F), lambda i: (i, 0))] * 2,
        out_shape=[jax.ShapeDtypeStruct((_N, F), jnp.float32)] * 2,
    )(xp, Wl, Wr)


def _edge1(xls, xrd, ea, We, attf, S8, T8, P128, blk=1000):
    def body(xls_ref, xrd_ref, ea_ref, we_ref, att_ref, s8_ref, t8_ref,
             p16_ref, ca_ref, cb_ref, dn_ref):
        ea_v = ea_ref[...]
        ef = ea_v[:, 0:1] * we_ref[0:1, :] + ea_v[:, 1:2] * we_ref[1:2, :]
        xlv = xls_ref[...]
        m = xlv + xrd_ref[...] + ef
        m = jnp.where(m > 0, m, 0.2 * m)
        lg = jnp.dot(m * att_ref[...], s8_ref[...],
                     preferred_element_type=jnp.float32, precision=lax.Precision.HIGHEST)
        ex8 = jnp.exp(lg)
        exb = jnp.dot(ex8, t8_ref[...], preferred_element_type=jnp.float32, precision=lax.Precision.HIGHEST)
        contrib = exb * xlv
        ca_ref[...] = contrib[:, :128]
        cb_ref[...] = contrib[:, 128:]
        dn_ref[...] = jnp.dot(ex8, p16_ref[...],
                              preferred_element_type=jnp.float32, precision=lax.Precision.HIGHEST)

    return _call(
        body,
        grid=(_E // blk,),
        in_specs=[
            pl.BlockSpec((blk, 256), lambda i: (i, 0)),
            pl.BlockSpec((blk, 256), lambda i: (i, 0)),
            pl.BlockSpec((blk, 2), lambda i: (i, 0)),
            pl.BlockSpec((2, 256), lambda i: (0, 0)),
            pl.BlockSpec((1, 256), lambda i: (0, 0)),
            pl.BlockSpec((256, 8), lambda i: (0, 0)),
            pl.BlockSpec((8, 256), lambda i: (0, 0)),
            pl.BlockSpec((8, 128), lambda i: (0, 0)),
        ],
        out_specs=[
            pl.BlockSpec((blk, 128), lambda i: (i, 0)),
            pl.BlockSpec((blk, 128), lambda i: (i, 0)),
            pl.BlockSpec((blk, 128), lambda i: (i, 0)),
        ],
        out_shape=[
            jax.ShapeDtypeStruct((_E, 128), jnp.float32),
            jax.ShapeDtypeStruct((_E, 128), jnp.float32),
            jax.ShapeDtypeStruct((_E, 128), jnp.float32),
        ],
    )(xls, xrd, ea, We, attf, S8, T8, P128)


def _node1(xl1, xr1, a0, a1, b0, b1, d0, d1, e0, e1, We, attf, S8, T8, P128,
           T128, b1f, Wl2, Wr2, blk=1000):
    def body(xl_ref, xr_ref, a0_ref, a1_ref, b0_ref, b1_ref, d0_ref, d1_ref,
             e0_ref, e1_ref, we_ref, att_ref, s8_ref, t8_ref, p16_ref,
             t16_ref, bias_ref, wl2_ref, wr2_ref, ot_ref):
        xlv = xl_ref[...]
        eacc = e0_ref[...] + e1_ref[...]
        cnt = jnp.maximum(eacc[:, 2:3], 1.0)
        la0 = eacc[:, 0:1] / cnt
        la1 = eacc[:, 1:2] / cnt
        efl = la0 * we_ref[0:1, :] + la1 * we_ref[1:2, :]
        ml = xlv + xr_ref[...] + efl
        ml = jnp.where(ml > 0, ml, 0.2 * ml)
        exl8 = jnp.exp(jnp.dot(ml * att_ref[...], s8_ref[...],
                               preferred_element_type=jnp.float32, precision=lax.Precision.HIGHEST))
        exlb = jnp.dot(exl8, t8_ref[...], preferred_element_type=jnp.float32, precision=lax.Precision.HIGHEST)
        den16 = d0_ref[...] + d1_ref[...] + jnp.dot(
            exl8, p16_ref[...], preferred_element_type=jnp.float32, precision=lax.Precision.HIGHEST)
        denb = jnp.dot(den16, t16_ref[...], preferred_element_type=jnp.float32, precision=lax.Precision.HIGHEST)
        num = jnp.concatenate(
            [a0_ref[...] + a1_ref[...], b0_ref[...] + b1_ref[...]], axis=1)
        num = num + exlb * xlv
        out = num / (denb + 1e-16) + bias_ref[...]
        h1 = jnp.where(out > 0, out, jnp.exp(out) - 1.0)
        ot_ref[...] = jnp.concatenate(
            [jnp.dot(h1, wl2_ref[...], preferred_element_type=jnp.float32, precision=lax.Precision.HIGHEST),
             jnp.dot(h1, wr2_ref[...], preferred_element_type=jnp.float32, precision=lax.Precision.HIGHEST)],
            axis=1)

    nb = lambda i: (i, 0)
    z = lambda i: (0, 0)
    return _call(
        body,
        grid=(_N // blk,),
        in_specs=[
            pl.BlockSpec((blk, 256), nb), pl.BlockSpec((blk, 256), nb),
            pl.BlockSpec((blk, 128), nb), pl.BlockSpec((blk, 128), nb),
            pl.BlockSpec((blk, 128), nb), pl.BlockSpec((blk, 128), nb),
            pl.BlockSpec((blk, 128), nb), pl.BlockSpec((blk, 128), nb),
            pl.BlockSpec((blk, 128), nb), pl.BlockSpec((blk, 128), nb),
            pl.BlockSpec((2, 256), z), pl.BlockSpec((1, 256), z),
            pl.BlockSpec((256, 8), z), pl.BlockSpec((8, 256), z),
            pl.BlockSpec((8, 128), z), pl.BlockSpec((128, 256), z),
            pl.BlockSpec((1, 256), z),
            pl.BlockSpec((256, 64), z), pl.BlockSpec((256, 64), z),
        ],
        out_specs=[pl.BlockSpec((blk, 128), nb)],
        out_shape=[jax.ShapeDtypeStruct((_N, 128), jnp.float32)],
    )(xl1, xr1, a0, a1, b0, b1, d0, d1, e0, e1, We, attf, S8, T8, P128, T128,
      b1f, Wl2, Wr2)[0]


def _edge2(xls, xrd, ea, We, attf, u64, blk=1000):
    def body(xls_ref, xrd_ref, ea_ref, we_ref, att_ref, u64_ref, c_ref):
        ea_v = ea_ref[...]
        ef = ea_v[:, 0:1] * we_ref[0:1, :] + ea_v[:, 1:2] * we_ref[1:2, :]
        xlv = xls_ref[:, 0:64]
        m = xlv + xrd_ref[:, 64:128] + ef
        m = jnp.where(m > 0, m, 0.2 * m)
        ex = jnp.exp(jnp.sum(m * att_ref[...], axis=1, keepdims=True))
        c_ref[...] = jnp.concatenate([ex * xlv, ex * u64_ref[...]], axis=1)

    nb = lambda i: (i, 0)
    z = lambda i: (0, 0)
    return _call(
        body,
        grid=(_E // blk,),
        in_specs=[
            pl.BlockSpec((blk, 128), nb), pl.BlockSpec((blk, 128), nb),
            pl.BlockSpec((blk, 2), nb),
            pl.BlockSpec((2, 64), z), pl.BlockSpec((1, 64), z),
            pl.BlockSpec((1, 64), z),
        ],
        out_specs=[pl.BlockSpec((blk, 128), nb)],
        out_shape=[jax.ShapeDtypeStruct((_E, 128), jnp.float32)],
    )(xls, xrd, ea, We, attf, u64)[0]


def _node2(t2, c0, c1, e0, e1, We, attf, b2f, blk=1000):
    def body(t_ref, c0_ref, c1_ref, e0_ref, e1_ref,
             we_ref, att_ref, bias_ref, oh_ref):
        xlv = t_ref[:, 0:64]
        eacc = e0_ref[...] + e1_ref[...]
        cnt = jnp.maximum(eacc[:, 2:3], 1.0)
        la0 = eacc[:, 0:1] / cnt
        la1 = eacc[:, 1:2] / cnt
        efl = la0 * we_ref[0:1, :] + la1 * we_ref[1:2, :]
        ml = xlv + t_ref[:, 64:128] + efl
        ml = jnp.where(ml > 0, ml, 0.2 * ml)
        exl = jnp.exp(jnp.sum(ml * att_ref[...], axis=1, keepdims=True))
        cacc = c0_ref[...] + c1_ref[...]
        dent = cacc[:, 64:65] + exl
        num = cacc[:, 0:64] + exl * xlv
        out = num / (dent + 1e-16) + bias_ref[...]
        oh_ref[...] = jnp.where(out > 0, out, jnp.exp(out) - 1.0)

    nb = lambda i: (i, 0)
    z = lambda i: (0, 0)
    return _call(
        body,
        grid=(_N // blk,),
        in_specs=[
            pl.BlockSpec((blk, 128), nb),
            pl.BlockSpec((blk, 128), nb), pl.BlockSpec((blk, 128), nb),
            pl.BlockSpec((blk, 128), nb), pl.BlockSpec((blk, 128), nb),
            pl.BlockSpec((2, 64), z), pl.BlockSpec((1, 64), z),
            pl.BlockSpec((1, 64), z),
        ],
        out_specs=[pl.BlockSpec((blk, 64), nb)],
        out_shape=[jax.ShapeDtypeStruct((_N, 64), jnp.float32)],
    )(t2, c0, c1, e0, e1, We, attf, b2f)[0]


def _sig(x):
    return 1.0 / (1.0 + jnp.exp(-x))


def _head(h2, bt, WihT, WhhT, bih, bhh, Wc1, bc1, Wc2p, bc2p):
    nchunk = _N // 1000

    def body(h2_ref, bt_ref, wih_ref, whh_ref, bih_ref, bhh_ref, wc1_ref,
             bc1_ref, wc2_ref, bc2_ref, o_ref, gs_ref, ct_ref, gi_ref):
        t = pl.program_id(0)

        @pl.when(t == 0)
        def _init():
            gs_ref[...] = jnp.zeros((_G, 64), jnp.float32)
            ct_ref[...] = jnp.zeros((_G, 1), jnp.float32)

        bb = bt_ref[0]
        gid = lax.broadcasted_iota(jnp.int32, (_G, 1000), 0)
        oh = (gid == bb).astype(jnp.float32)
        gs_ref[...] += jnp.dot(oh, h2_ref[...],
                               preferred_element_type=jnp.float32, precision=lax.Precision.HIGHEST)
        ct_ref[...] += jnp.sum(oh, axis=1, keepdims=True)

        @pl.when(t == nchunk - 1)
        def _finish():
            g = gs_ref[...] / jnp.maximum(ct_ref[...], 1.0)
            gi_ref[...] = jnp.dot(
                g, wih_ref[...], preferred_element_type=jnp.float32, precision=lax.Precision.HIGHEST
            ) + bih_ref[...]

            def gru(i, h):
                gi = gi_ref[pl.ds(i, 1), :]
                gh = jnp.dot(h, whh_ref[...],
                             preferred_element_type=jnp.float32, precision=lax.Precision.HIGHEST) + bhh_ref[...]
                r = _sig(gi[:, 0:64] + gh[:, 0:64])
                zz = _sig(gi[:, 64:128] + gh[:, 64:128])
                nt = jnp.tanh(gi[:, 128:192] + r * gh[:, 128:192])
                return (1.0 - zz) * nt + zz * h

            h = lax.fori_loop(0, _G, gru, jnp.zeros((1, 64), jnp.float32))
            z1 = jnp.maximum(
                jnp.dot(h, wc1_ref[...], preferred_element_type=jnp.float32, precision=lax.Precision.HIGHEST)
                + bc1_ref[...], 0.0)
            o_ref[...] = _sig(
                jnp.dot(z1, wc2_ref[...], preferred_element_type=jnp.float32, precision=lax.Precision.HIGHEST)
                + bc2_ref[...])

    nb = lambda i: (i, 0)
    z = lambda i: (0, 0)
    return _call(
        body,
        grid=(nchunk,),
        in_specs=[
            pl.BlockSpec((1000, 64), nb),
            pl.BlockSpec((1, 1, 1000), lambda i: (i, 0, 0)),
            pl.BlockSpec((64, 192), z), pl.BlockSpec((64, 192), z),
            pl.BlockSpec((1, 192), z), pl.BlockSpec((1, 192), z),
            pl.BlockSpec((64, 32), z), pl.BlockSpec((1, 32), z),
            pl.BlockSpec((32, 8), z), pl.BlockSpec((1, 8), z),
        ],
        out_specs=[pl.BlockSpec((1, 8), z)],
        out_shape=[jax.ShapeDtypeStruct((1, 8), jnp.float32)],
        scratch_shapes=[
            pltpu.VMEM((_G, 64), jnp.float32),
            pltpu.VMEM((_G, 1), jnp.float32),
            pltpu.VMEM((_G, 192), jnp.float32),
        ],
    )(h2, bt, WihT, WhhT, bih, bhh, Wc1, bc1, Wc2p, bc2p)[0]


_HSEL = np.arange(256) // 64
_S8 = np.zeros((256, 8), np.float32)
_S8[np.arange(256), _HSEL] = 1.0
_T8 = np.ascontiguousarray(_S8.T)
_P128 = np.zeros((8, 128), np.float32)
for _i in range(4):
    _P128[_i, _i] = 1.0
_T128 = np.zeros((128, 256), np.float32)
for _h in range(4):
    _T128[_h, _h * 64:(_h + 1) * 64] = 1.0
_U64 = np.zeros((1, 64), np.float32)
_U64[0, 0] = 1.0


def kernel(x, edge_index, edge_attr, batch, Wl1, Wr1, We1, att1, b1, Wl2,
           Wr2, We2, att2, b2, Wih, Whh, bih, bhh, Wc1, bc1, Wc2, bc2):
    f32 = jnp.float32
    src = edge_index[0]
    dst = edge_index[1]

    xp = jnp.pad(x, ((0, 0), (0, 3)))
    Wl1p = jnp.pad(Wl1, ((0, 3), (0, 0)))
    Wr1p = jnp.pad(Wr1, ((0, 3), (0, 0)))
    attf1 = att1.reshape(1, 256)
    attf2 = att2.reshape(1, 64)
    b1f = b1.reshape(1, 256)
    b2f = b2.reshape(1, 64)
    S8 = jnp.asarray(_S8)
    T8 = jnp.asarray(_T8)
    P128 = jnp.asarray(_P128)
    T128 = jnp.asarray(_T128)
    u64 = jnp.asarray(_U64)
    z128 = jnp.zeros((_NPAD, 128), f32)

    rows_ea = jnp.concatenate(
        [edge_attr, jnp.ones((_E, 1), f32), jnp.zeros((_E, 125), f32)], axis=1)
    eacc = _scatter_add_sc(rows_ea, dst, z128, 128)

    xl1, xr1 = _proj_pair(xp, Wl1p, Wr1p, 8, 256)
    xls1, xrd1 = _gather_pair_sc(xl1, xr1, src, dst, 256)
    cA, cB, dn1 = _edge1(xls1, xrd1, edge_attr, We1, attf1, S8, T8, P128)
    numA = _scatter_add_sc(cA, dst, z128, 128)
    numB = _scatter_add_sc(cB, dst, z128, 128)
    den1 = _scatter_add_sc(dn1, dst, z128, 128)
    t2 = _node1(xl1, xr1, numA[0], numA[1], numB[0], numB[1], den1[0],
                den1[1], eacc[0], eacc[1], We1, attf1, S8, T8, P128, T128,
                b1f, Wl2, Wr2)

    xls2, xrd2 = _gather_pair_sc(t2, t2, src, dst, 128)
    c2 = _edge2(xls2, xrd2, edge_attr, We2, attf2, u64)
    acc2 = _scatter_add_sc(c2, dst, z128, 128)
    h2 = _node2(t2, acc2[0], acc2[1], eacc[0], eacc[1], We2, attf2, b2f)

    WihT = Wih.T
    WhhT = Whh.T
    Wc2p = jnp.pad(Wc2, ((0, 0), (0, 7)))
    bc2p = jnp.pad(bc2.reshape(1, 1), ((0, 0), (0, 7)))
    out8 = _head(h2, batch.reshape(10, 1, 1000), WihT, WhhT, bih.reshape(1, 192),
                 bhh.reshape(1, 192), Wc1, bc1.reshape(1, 32), Wc2p, bc2p)
    return out8[0:1, 0:1]


# speedup vs baseline: 16.4572x; 1.1693x over previous
import functools

import jax
import jax.numpy as jnp
import numpy as np
from jax import lax
from jax.experimental import pallas as pl
from jax.experimental.pallas import tpu as pltpu
from jax.experimental.pallas import tpu_sc as plsc

_N = 10000
_E = 160000
_G = 64
_NC = 2
_NS = 16
_NW = _NC * _NS
_EPW = _E // _NW
_C = 40
_NPAD = 10240
_RPT = _NPAD // _NS

_call = pl.pallas_call


def _sc_mesh():
    return plsc.VectorSubcoreMesh(core_axis_name="c", subcore_axis_name="s")


_CHUNKS = [(i * 120, 120) for i in range(41)] + [(4920, 80)]
_NBUF = 3


def _gather_pair_sc(ta, tb, src, dst, F):
    def body(ta_h, tb_h, src_h, dst_h, oa_h, ob_h, idxs_v, idxd_v,
             b0, b1, b2, gs0, gs1, gs2, ws0, ws1, ws2):
        c = lax.axis_index("c")
        s = lax.axis_index("s")
        base = (s * _NC + c) * _EPW
        pltpu.sync_copy(src_h.at[pl.ds(base, _EPW)], idxs_v)
        pltpu.sync_copy(dst_h.at[pl.ds(base, _EPW)], idxd_v)
        bufs = (b0, b1, b2)
        gsems = (gs0, gs1, gs2)
        wsems = (ws0, ws1, ws2)

        steps = []
        for tab, out, idxv in ((ta_h, oa_h, idxs_v), (tb_h, ob_h, idxd_v)):
            for off, sz in _CHUNKS:
                steps.append((tab, out, idxv, off, sz))

        pend_g = [None] * _NBUF
        pend_w = [None] * _NBUF
        meta = [None] * _NBUF
        for j, (tab, out, idxv, off, sz) in enumerate(steps):
            b = j % _NBUF
            if pend_w[b] is not None:
                pend_w[b].wait()
                pend_w[b] = None
            pend_g[b] = pltpu.async_copy(
                tab.at[idxv.at[pl.ds(off, sz)]],
                bufs[b].at[pl.ds(0, sz)], gsems[b])
            meta[b] = (out, off, sz)
            bp = (j + 1) % _NBUF
            if j >= _NBUF - 1 and pend_g[bp] is not None:
                pend_g[bp].wait()
                pend_g[bp] = None
                outp, offp, szp = meta[bp]
                pend_w[bp] = pltpu.async_copy(
                    bufs[bp].at[pl.ds(0, szp)],
                    outp.at[pl.ds(base + offp, szp)], wsems[bp])
        for b in range(_NBUF):
            if pend_g[b] is not None:
                pend_g[b].wait()
                outp, offp, szp = meta[b]
                pend_w[b] = pltpu.async_copy(
                    bufs[b].at[pl.ds(0, szp)],
                    outp.at[pl.ds(base + offp, szp)], wsems[b])
            if pend_w[b] is not None:
                pend_w[b].wait()

    k = pl.kernel(
        body,
        out_type=[jax.ShapeDtypeStruct((_E, F), jnp.float32)] * 2,
        mesh=_sc_mesh(),
        scratch_types=[
            pltpu.VMEM((_EPW,), jnp.int32),
            pltpu.VMEM((_EPW,), jnp.int32),
            pltpu.VMEM((120, F), jnp.float32),
            pltpu.VMEM((120, F), jnp.float32),
            pltpu.VMEM((120, F), jnp.float32),
            pltpu.SemaphoreType.DMA, pltpu.SemaphoreType.DMA,
            pltpu.SemaphoreType.DMA, pltpu.SemaphoreType.DMA,
            pltpu.SemaphoreType.DMA, pltpu.SemaphoreType.DMA,
        ],
    )
    return k(ta, tb, src, dst)


def _scatter_add_sc(rows, dst, zeros, D):
    def body(rows_h, dst_h, z_h, out_h, i0, i1, it_, r0, r1, s0, s1, acc_sh):
        c = lax.axis_index("c")
        s = lax.axis_index("s")
        base = (s * _NC + c) * _EPW
        pltpu.sync_copy(z_h.at[pl.ds(s * _RPT, _RPT)],
                        acc_sh.at[pl.ds(s * _RPT, _RPT)])
        plsc.subcore_barrier()

        ibufs = (i0, i1)
        rbufs = (r0, r1)
        sems = (s0, s1)
        nch = len(_CHUNKS)

        def issue(k):
            off, sz = _CHUNKS[k]
            b = k % 2
            ib = it_ if sz != 120 else ibufs[b]
            icp = pltpu.async_copy(dst_h.at[pl.ds(base + off, sz)], ib,
                                   sems[b])
            rcp = pltpu.async_copy(rows_h.at[pl.ds(base + off, sz)],
                                   rbufs[b].at[pl.ds(0, sz)], sems[b])
            return icp, rcp, ib, sz

        pend = [None, None]
        pend[0] = issue(0)
        for k in range(nch):
            b = k % 2
            if k + 1 < nch:
                pend[(k + 1) % 2] = issue(k + 1)
            icp, rcp, ib, sz = pend[b]
            icp.wait()
            rcp.wait()
            pltpu.sync_copy(rbufs[b].at[pl.ds(0, sz)], acc_sh.at[ib],
                            add=True)

        plsc.subcore_barrier()
        pltpu.sync_copy(acc_sh.at[pl.ds(s * _RPT, _RPT)],
                        out_h.at[c, pl.ds(s * _RPT, _RPT)])

    k = pl.kernel(
        body,
        out_type=jax.ShapeDtypeStruct((2, _NPAD, D), jnp.float32),
        mesh=_sc_mesh(),
        scratch_types=[
            pltpu.VMEM((120,), jnp.int32),
            pltpu.VMEM((120,), jnp.int32),
            pltpu.VMEM((80,), jnp.int32),
            pltpu.VMEM((120, D), jnp.float32),
            pltpu.VMEM((120, D), jnp.float32),
            pltpu.SemaphoreType.DMA, pltpu.SemaphoreType.DMA,
            pltpu.VMEM_SHARED((_NPAD, D), jnp.float32),
        ],
    )
    out = k(rows, dst, zeros)
    return out[:, :_N]


def _proj_pair(xp, Wl, Wr, K, F, blk=1000):
    def body(x_ref, wl_ref, wr_ref, ol_ref, or_ref):
        xv = x_ref[...]
        ol_ref[...] = jnp.dot(xv, wl_ref[...], preferred_element_type=jnp.float32, precision=lax.Precision.HIGHEST)
        or_ref[...] = jnp.dot(xv, wr_ref[...], preferred_element_type=jnp.float32, precision=lax.Precision.HIGHEST)

    return _call(
        body,
        grid=(_N // blk,),
        in_specs=[
            pl.BlockSpec((blk, K), lambda i: (i, 0)),
            pl.BlockSpec((K, F), lambda i: (0, 0)),
            pl.BlockSpec((K, F), lambda i: (0, 0)),
        ],
        out_specs=[pl.BlockSpec((blk, F), lambda i: (i, 0))] * 2,
        out_shape=[jax.ShapeDtypeStruct((_N, F), jnp.float32)] * 2,
    )(xp, Wl, Wr)


def _edge1(xls, xrd, ea, We, attf, S8, T8, P128, blk=1000):
    def body(xls_ref, xrd_ref, ea_ref, we_ref, att_ref, s8_ref, t8_ref,
             p16_ref, ca_ref, cb_ref, dn_ref):
        ea_v = ea_ref[...]
        ef = ea_v[:, 0:1] * we_ref[0:1, :] + ea_v[:, 1:2] * we_ref[1:2, :]
        xlv = xls_ref[...]
        m = xlv + xrd_ref[...] + ef
        m = jnp.where(m > 0, m, 0.2 * m)
        lg = jnp.dot(m * att_ref[...], s8_ref[...],
                     preferred_element_type=jnp.float32, precision=lax.Precision.HIGHEST)
        ex8 = jnp.exp(lg)
        exb = jnp.dot(ex8, t8_ref[...], preferred_element_type=jnp.float32, precision=lax.Precision.HIGHEST)
        contrib = exb * xlv
        ca_ref[...] = contrib[:, :128]
        cb_ref[...] = contrib[:, 128:]
        dn_ref[...] = jnp.concatenate(
            [ex8, ea_v, jnp.ones((ex8.shape[0], 1), jnp.float32),
             jnp.zeros((ex8.shape[0], 117), jnp.float32)], axis=1)

    return _call(
        body,
        grid=(_E // blk,),
        in_specs=[
            pl.BlockSpec((blk, 256), lambda i: (i, 0)),
            pl.BlockSpec((blk, 256), lambda i: (i, 0)),
            pl.BlockSpec((blk, 2), lambda i: (i, 0)),
            pl.BlockSpec((2, 256), lambda i: (0, 0)),
            pl.BlockSpec((1, 256), lambda i: (0, 0)),
            pl.BlockSpec((256, 8), lambda i: (0, 0)),
            pl.BlockSpec((8, 256), lambda i: (0, 0)),
            pl.BlockSpec((8, 128), lambda i: (0, 0)),
        ],
        out_specs=[
            pl.BlockSpec((blk, 128), lambda i: (i, 0)),
            pl.BlockSpec((blk, 128), lambda i: (i, 0)),
            pl.BlockSpec((blk, 128), lambda i: (i, 0)),
        ],
        out_shape=[
            jax.ShapeDtypeStruct((_E, 128), jnp.float32),
            jax.ShapeDtypeStruct((_E, 128), jnp.float32),
            jax.ShapeDtypeStruct((_E, 128), jnp.float32),
        ],
    )(xls, xrd, ea, We, attf, S8, T8, P128)


def _node1(xl1, xr1, a0, a1, b0, b1, d0, d1, We, attf, S8, T8, P128,
           T128, b1f, Wl2, Wr2, blk=1000):
    def body(xl_ref, xr_ref, a0_ref, a1_ref, b0_ref, b1_ref, d0_ref, d1_ref,
             we_ref, att_ref, s8_ref, t8_ref, p16_ref,
             t16_ref, bias_ref, wl2_ref, wr2_ref, ot_ref):
        xlv = xl_ref[...]
        dacc = d0_ref[...] + d1_ref[...]
        cnt = jnp.maximum(dacc[:, 10:11], 1.0)
        la0 = dacc[:, 8:9] / cnt
        la1 = dacc[:, 9:10] / cnt
        efl = la0 * we_ref[0:1, :] + la1 * we_ref[1:2, :]
        ml = xlv + xr_ref[...] + efl
        ml = jnp.where(ml > 0, ml, 0.2 * ml)
        exl8 = jnp.exp(jnp.dot(ml * att_ref[...], s8_ref[...],
                               preferred_element_type=jnp.float32, precision=lax.Precision.HIGHEST))
        exlb = jnp.dot(exl8, t8_ref[...], preferred_element_type=jnp.float32, precision=lax.Precision.HIGHEST)
        den16 = dacc + jnp.dot(
            exl8, p16_ref[...], preferred_element_type=jnp.float32, precision=lax.Precision.HIGHEST)
        denb = jnp.dot(den16, t16_ref[...], preferred_element_type=jnp.float32, precision=lax.Precision.HIGHEST)
        num = jnp.concatenate(
            [a0_ref[...] + a1_ref[...], b0_ref[...] + b1_ref[...]], axis=1)
        num = num + exlb * xlv
        out = num / (denb + 1e-16) + bias_ref[...]
        h1 = jnp.where(out > 0, out, jnp.exp(out) - 1.0)
        ot_ref[...] = jnp.concatenate(
            [jnp.dot(h1, wl2_ref[...], preferred_element_type=jnp.float32, precision=lax.Precision.HIGHEST),
             jnp.dot(h1, wr2_ref[...], preferred_element_type=jnp.float32, precision=lax.Precision.HIGHEST)],
            axis=1)

    nb = lambda i: (i, 0)
    z = lambda i: (0, 0)
    return _call(
        body,
        grid=(_N // blk,),
        in_specs=[
            pl.BlockSpec((blk, 256), nb), pl.BlockSpec((blk, 256), nb),
            pl.BlockSpec((blk, 128), nb), pl.BlockSpec((blk, 128), nb),
            pl.BlockSpec((blk, 128), nb), pl.BlockSpec((blk, 128), nb),
            pl.BlockSpec((blk, 128), nb), pl.BlockSpec((blk, 128), nb),
            pl.BlockSpec((2, 256), z), pl.BlockSpec((1, 256), z),
            pl.BlockSpec((256, 8), z), pl.BlockSpec((8, 256), z),
            pl.BlockSpec((8, 128), z), pl.BlockSpec((128, 256), z),
            pl.BlockSpec((1, 256), z),
            pl.BlockSpec((256, 64), z), pl.BlockSpec((256, 64), z),
        ],
        out_specs=[pl.BlockSpec((blk, 128), nb)],
        out_shape=[jax.ShapeDtypeStruct((_N, 128), jnp.float32)],
    )(xl1, xr1, a0, a1, b0, b1, d0, d1, We, attf, S8, T8, P128, T128,
      b1f, Wl2, Wr2)[0]


def _edge2(xls, xrd, ea, We, attf, u64, blk=1000):
    def body(xls_ref, xrd_ref, ea_ref, we_ref, att_ref, u64_ref, c_ref):
        ea_v = ea_ref[...]
        ef = ea_v[:, 0:1] * we_ref[0:1, :] + ea_v[:, 1:2] * we_ref[1:2, :]
        xlv = xls_ref[:, 0:64]
        m = xlv + xrd_ref[:, 64:128] + ef
        m = jnp.where(m > 0, m, 0.2 * m)
        ex = jnp.exp(jnp.sum(m * att_ref[...], axis=1, keepdims=True))
        c_ref[...] = jnp.concatenate([ex * xlv, ex * u64_ref[...]], axis=1)

    nb = lambda i: (i, 0)
    z = lambda i: (0, 0)
    return _call(
        body,
        grid=(_E // blk,),
        in_specs=[
            pl.BlockSpec((blk, 128), nb), pl.BlockSpec((blk, 128), nb),
            pl.BlockSpec((blk, 2), nb),
            pl.BlockSpec((2, 64), z), pl.BlockSpec((1, 64), z),
            pl.BlockSpec((1, 64), z),
        ],
        out_specs=[pl.BlockSpec((blk, 128), nb)],
        out_shape=[jax.ShapeDtypeStruct((_E, 128), jnp.float32)],
    )(xls, xrd, ea, We, attf, u64)[0]


def _node2(t2, c0, c1, d0, d1, We, attf, b2f, blk=1000):
    def body(t_ref, c0_ref, c1_ref, d0_ref, d1_ref,
             we_ref, att_ref, bias_ref, oh_ref):
        xlv = t_ref[:, 0:64]
        dacc = d0_ref[...] + d1_ref[...]
        cnt = jnp.maximum(dacc[:, 10:11], 1.0)
        la0 = dacc[:, 8:9] / cnt
        la1 = dacc[:, 9:10] / cnt
        efl = la0 * we_ref[0:1, :] + la1 * we_ref[1:2, :]
        ml = xlv + t_ref[:, 64:128] + efl
        ml = jnp.where(ml > 0, ml, 0.2 * ml)
        exl = jnp.exp(jnp.sum(ml * att_ref[...], axis=1, keepdims=True))
        cacc = c0_ref[...] + c1_ref[...]
        dent = cacc[:, 64:65] + exl
        num = cacc[:, 0:64] + exl * xlv
        out = num / (dent + 1e-16) + bias_ref[...]
        oh_ref[...] = jnp.where(out > 0, out, jnp.exp(out) - 1.0)

    nb = lambda i: (i, 0)
    z = lambda i: (0, 0)
    return _call(
        body,
        grid=(_N // blk,),
        in_specs=[
            pl.BlockSpec((blk, 128), nb),
            pl.BlockSpec((blk, 128), nb), pl.BlockSpec((blk, 128), nb),
            pl.BlockSpec((blk, 128), nb), pl.BlockSpec((blk, 128), nb),
            pl.BlockSpec((2, 64), z), pl.BlockSpec((1, 64), z),
            pl.BlockSpec((1, 64), z),
        ],
        out_specs=[pl.BlockSpec((blk, 64), nb)],
        out_shape=[jax.ShapeDtypeStruct((_N, 64), jnp.float32)],
    )(t2, c0, c1, d0, d1, We, attf, b2f)[0]


def _sig(x):
    return 1.0 / (1.0 + jnp.exp(-x))


def _head(h2, bt, WihT, WhhT, bih, bhh, Wc1, bc1, Wc2p, bc2p):
    nchunk = _N // 1000

    def body(h2_ref, bt_ref, wih_ref, whh_ref, bih_ref, bhh_ref, wc1_ref,
             bc1_ref, wc2_ref, bc2_ref, o_ref, gs_ref, ct_ref, gi_ref):
        t = pl.program_id(0)

        @pl.when(t == 0)
        def _init():
            gs_ref[...] = jnp.zeros((_G, 64), jnp.float32)
            ct_ref[...] = jnp.zeros((_G, 1), jnp.float32)

        bb = bt_ref[0]
        gid = lax.broadcasted_iota(jnp.int32, (_G, 1000), 0)
        oh = (gid == bb).astype(jnp.float32)
        gs_ref[...] += jnp.dot(oh, h2_ref[...],
                               preferred_element_type=jnp.float32, precision=lax.Precision.HIGHEST)
        ct_ref[...] += jnp.sum(oh, axis=1, keepdims=True)

        @pl.when(t == nchunk - 1)
        def _finish():
            g = gs_ref[...] / jnp.maximum(ct_ref[...], 1.0)
            gi_ref[...] = jnp.dot(
                g, wih_ref[...], preferred_element_type=jnp.float32, precision=lax.Precision.HIGHEST
            ) + bih_ref[...]

            def gru(i, h):
                gi = gi_ref[pl.ds(i, 1), :]
                gh = jnp.dot(h, whh_ref[...],
                             preferred_element_type=jnp.float32, precision=lax.Precision.HIGHEST) + bhh_ref[...]
                r = _sig(gi[:, 0:64] + gh[:, 0:64])
                zz = _sig(gi[:, 64:128] + gh[:, 64:128])
                nt = jnp.tanh(gi[:, 128:192] + r * gh[:, 128:192])
                return (1.0 - zz) * nt + zz * h

            h = lax.fori_loop(0, _G, gru, jnp.zeros((1, 64), jnp.float32))
            z1 = jnp.maximum(
                jnp.dot(h, wc1_ref[...], preferred_element_type=jnp.float32, precision=lax.Precision.HIGHEST)
                + bc1_ref[...], 0.0)
            o_ref[...] = _sig(
                jnp.dot(z1, wc2_ref[...], preferred_element_type=jnp.float32, precision=lax.Precision.HIGHEST)
                + bc2_ref[...])

    nb = lambda i: (i, 0)
    z = lambda i: (0, 0)
    return _call(
        body,
        grid=(nchunk,),
        in_specs=[
            pl.BlockSpec((1000, 64), nb),
            pl.BlockSpec((1, 1, 1000), lambda i: (i, 0, 0)),
            pl.BlockSpec((64, 192), z), pl.BlockSpec((64, 192), z),
            pl.BlockSpec((1, 192), z), pl.BlockSpec((1, 192), z),
            pl.BlockSpec((64, 32), z), pl.BlockSpec((1, 32), z),
            pl.BlockSpec((32, 8), z), pl.BlockSpec((1, 8), z),
        ],
        out_specs=[pl.BlockSpec((1, 8), z)],
        out_shape=[jax.ShapeDtypeStruct((1, 8), jnp.float32)],
        scratch_shapes=[
            pltpu.VMEM((_G, 64), jnp.float32),
            pltpu.VMEM((_G, 1), jnp.float32),
            pltpu.VMEM((_G, 192), jnp.float32),
        ],
    )(h2, bt, WihT, WhhT, bih, bhh, Wc1, bc1, Wc2p, bc2p)[0]


_HSEL = np.arange(256) // 64
_S8 = np.zeros((256, 8), np.float32)
_S8[np.arange(256), _HSEL] = 1.0
_T8 = np.ascontiguousarray(_S8.T)
_P128 = np.zeros((8, 128), np.float32)
for _i in range(4):
    _P128[_i, _i] = 1.0
_T128 = np.zeros((128, 256), np.float32)
for _h in range(4):
    _T128[_h, _h * 64:(_h + 1) * 64] = 1.0
_U64 = np.zeros((1, 64), np.float32)
_U64[0, 0] = 1.0


def kernel(x, edge_index, edge_attr, batch, Wl1, Wr1, We1, att1, b1, Wl2,
           Wr2, We2, att2, b2, Wih, Whh, bih, bhh, Wc1, bc1, Wc2, bc2):
    f32 = jnp.float32
    src = edge_index[0]
    dst = edge_index[1]

    xp = jnp.pad(x, ((0, 0), (0, 3)))
    Wl1p = jnp.pad(Wl1, ((0, 3), (0, 0)))
    Wr1p = jnp.pad(Wr1, ((0, 3), (0, 0)))
    attf1 = att1.reshape(1, 256)
    attf2 = att2.reshape(1, 64)
    b1f = b1.reshape(1, 256)
    b2f = b2.reshape(1, 64)
    S8 = jnp.asarray(_S8)
    T8 = jnp.asarray(_T8)
    P128 = jnp.asarray(_P128)
    T128 = jnp.asarray(_T128)
    u64 = jnp.asarray(_U64)
    z128 = jnp.zeros((_NPAD, 128), f32)

    rows_ea = jnp.concatenate(
        [edge_attr, jnp.ones((_E, 1), f32), jnp.zeros((_E, 125), f32)], axis=1)
    eacc = _scatter_add_sc(rows_ea, dst, z128, 128)

    xl1, xr1 = _proj_pair(xp, Wl1p, Wr1p, 8, 256)
    xls1, xrd1 = _gather_pair_sc(xl1, xr1, src, dst, 256)
    cA, cB, dn1 = _edge1(xls1, xrd1, edge_attr, We1, attf1, S8, T8, P128)
    numA = _scatter_add_sc(cA, dst, z128, 128)
    numB = _scatter_add_sc(cB, dst, z128, 128)
    den1 = _scatter_add_sc(dn1, dst, z128, 128)
    t2 = _node1(xl1, xr1, numA[0], numA[1], numB[0], numB[1], den1[0],
                den1[1], We1, attf1, S8, T8, P128, T128,
                b1f, Wl2, Wr2)

    xls2, xrd2 = _gather_pair_sc(t2, t2, src, dst, 128)
    c2 = _edge2(xls2, xrd2, edge_attr, We2, attf2, u64)
    acc2 = _scatter_add_sc(c2, dst, z128, 128)
    h2 = _node2(t2, acc2[0], acc2[1], den1[0], den1[1], We2, attf2, b2f)

    WihT = Wih.T
    WhhT = Whh.T
    Wc2p = jnp.pad(Wc2, ((0, 0), (0, 7)))
    bc2p = jnp.pad(bc2.reshape(1, 1), ((0, 0), (0, 7)))
    out8 = _head(h2, batch.reshape(10, 1, 1000), WihT, WhhT, bih.reshape(1, 192),
                 bhh.reshape(1, 192), Wc1, bc1.reshape(1, 32), Wc2p, bc2p)
    return out8[0:1, 0:1]
